# baseline (speedup 1.0000x reference)
import functools
import jax
import jax.numpy as jnp
from jax import lax
from jax.experimental import pallas as pl
from jax.experimental.pallas import tpu as pltpu
from jax.experimental.pallas import tpu_sc as plsc

D = 64
SCALE = 8.0
NW = 32
VOCAB_SIZE = 1_000_000
BLK = 128
NBLK = VOCAB_SIZE // BLK
TAIL = VOCAB_SIZE - NBLK * BLK
K1_RING = 4
K2_RING = 4


def _iota16():
    return lax.iota(jnp.int32, 16)


@functools.partial(
    pl.kernel,
    out_type=jax.ShapeDtypeStruct((VOCAB_SIZE // 2, 2 * D), jnp.float32),
    mesh=plsc.VectorSubcoreMesh(core_axis_name="c", subcore_axis_name="s"),
    scratch_types=[
        [pltpu.VMEM((D, BLK), jnp.float32) for _ in range(K1_RING)],
        [pltpu.VMEM((BLK // 2, 2 * D), jnp.float32) for _ in range(2)],
        [pltpu.SemaphoreType.DMA for _ in range(K1_RING)],
        [pltpu.SemaphoreType.DMA for _ in range(2)],
    ],
    compiler_params=pltpu.CompilerParams(
        use_tc_tiling_on_sc=True, needs_layout_passes=False
    ),
)
def _transpose_lut(lutT_hbm, tail_hbm, scr_hbm, bufs, obufs, ssems, osems):
    wid = lax.axis_index("s") * 2 + lax.axis_index("c")
    per = NBLK // NW
    rem = NBLK - per * NW
    cnt = jnp.where(wid < rem, per + 1, per)
    lo = wid * per + jnp.minimum(wid, rem)
    riota = [_iota16() + 16 * m for m in range(4)]

    def fire_stage(u, b):
        pltpu.async_copy(
            lutT_hbm.at[:, pl.ds(u * BLK, BLK)], bufs[b], ssems[b]
        )

    def wait_stage(u, b):
        pltpu.make_async_copy(
            lutT_hbm.at[:, pl.ds(u * BLK, BLK)], bufs[b], ssems[b]
        ).wait()

    for p in range(K1_RING - 1):
        fire_stage(lo + p, p)

    n_slots = -(-(per + 1) // K1_RING) * K1_RING

    def body(i, carry):
        for b in range(K1_RING):
            u = i * K1_RING + b

            @pl.when(u < cnt)
            def _():
                blk = lo + u

                @pl.when(u + (K1_RING - 1) < cnt)
                def _():
                    fire_stage(blk + (K1_RING - 1), (b + K1_RING - 1) % K1_RING)

                wait_stage(blk, b)
                ob = b % 2

                @pl.when(u >= 2)
                def _():
                    pltpu.make_async_copy(
                        obufs[ob], scr_hbm.at[pl.ds(0, BLK // 2)], osems[ob]
                    ).wait()

                obuf = obufs[ob]
                buf = bufs[b]

                def trans_body(q, c2):
                    c0 = riota[0] * 0 + 2 * q
                    for k in range(8):
                        v = plsc.load_gather(buf, [riota[k % 4], c0 + (k // 4)])
                        obuf[q, pl.ds(16 * k, 16)] = v * SCALE
                    return c2

                lax.fori_loop(0, BLK // 2, trans_body, 0)
                pltpu.async_copy(
                    obuf, scr_hbm.at[pl.ds(blk * (BLK // 2), BLK // 2)], osems[ob]
                )
        return carry

    lax.fori_loop(0, n_slots // K1_RING, body, 0)
    for ob in range(2):
        pltpu.make_async_copy(
            obufs[ob], scr_hbm.at[pl.ds(0, BLK // 2)], osems[ob]
        ).wait()

    @pl.when(wid == NW - 1)
    def _():
        tb = obufs[0].at[pl.ds(0, TAIL // 2)]
        pltpu.sync_copy(tail_hbm, tb)

        def tail_body(q, c2):
            for k in range(8):
                sl = pl.ds(16 * k, 16)
                tb[q, sl] = tb[q, sl] * SCALE
            return c2

        lax.fori_loop(0, TAIL // 2, tail_body, 0)
        pltpu.sync_copy(tb, scr_hbm.at[pl.ds(NBLK * (BLK // 2), TAIL // 2)])


def _make_gather(b_total, seq):
    rows_per_worker = b_total // NW

    @functools.partial(
        pl.kernel,
        out_type=jax.ShapeDtypeStruct((seq, 8, b_total // 128, 8 * 128), jnp.float32),
        mesh=plsc.VectorSubcoreMesh(core_axis_name="c", subcore_axis_name="s"),
        scratch_types=[
            pltpu.VMEM((rows_per_worker, seq), jnp.int32),
            pltpu.VMEM((seq, rows_per_worker), jnp.int32),
            [pltpu.VMEM((rows_per_worker, D), jnp.float32) for _ in range(K2_RING)],
            [pltpu.VMEM((8, 8 * 128), jnp.float32) for _ in range(K2_RING)],
            [pltpu.SemaphoreType.DMA for _ in range(K2_RING)],
            [pltpu.SemaphoreType.DMA for _ in range(K2_RING)],
        ],
        compiler_params=pltpu.CompilerParams(
            use_tc_tiling_on_sc=False, needs_layout_passes=False
        ),
    )
    def k(scr_hbm, x_hbm, out_hbm, xbuf, idxT, gbufs, tbufs, gsems, osems):
        wid = lax.axis_index("s") * 2 + lax.axis_index("c")
        row0 = wid * rows_per_worker
        riota = [_iota16() + 16 * m for m in range(8)]

        pltpu.sync_copy(x_hbm.at[pl.ds(row0, rows_per_worker)], xbuf)

        def xt_body(q, c2):
            c0 = riota[0] * 0 + q
            for k in range(8):
                v = plsc.load_gather(xbuf, [riota[k], c0])
                idxT[q, pl.ds(16 * k, 16)] = v
            return c2

        lax.fori_loop(0, seq, xt_body, 0)

        def fire_gather(c, b):
            pltpu.async_copy(scr_hbm.at[idxT.at[c]], gbufs[b], gsems[b])

        def drain_gather(c, b):
            pltpu.make_async_copy(
                scr_hbm.at[idxT.at[c]], gbufs[b], gsems[b]
            ).wait()

        PF = 2
        for b in range(PF):
            fire_gather(b, b)

        def body(i, carry):
            for b in range(K2_RING):
                c = i * K2_RING + b
                b_pre = (b + PF) % K2_RING

                @pl.when(c + PF <= seq - 1)
                def _():
                    fire_gather(c + PF, b_pre)

                drain_gather(c, b)
                gbuf = gbufs[b]
                tbuf = tbufs[b]

                @pl.when(c >= K2_RING)
                def _():
                    pltpu.make_async_copy(
                        tbufs[b], out_hbm.at[0, :, wid], osems[b]
                    ).wait()

                def ext_body(dblk, c2):
                    for d_in in range(8):
                        dcol = riota[0] * 0 + (dblk * 8 + d_in)
                        for g in range(8):
                            v = plsc.load_gather(gbuf, [riota[g], dcol])
                            tbuf[dblk, pl.ds(128 * d_in + 16 * g, 16)] = v
                    return c2

                lax.fori_loop(0, 8, ext_body, 0)
                pltpu.async_copy(tbuf, out_hbm.at[c, :, wid], osems[b])
            return carry

        lax.fori_loop(0, seq // K2_RING, body, 0)
        for b in range(K2_RING):
            pltpu.make_async_copy(
                tbufs[b], out_hbm.at[0, :, wid], osems[b]
            ).wait()

    return k


@jax.jit
def _embed(x, lut):
    b_total, seq = x.shape
    lutT = jnp.swapaxes(lut, 0, 1)
    tail = lut[NBLK * BLK :].reshape(TAIL // 2, 2 * D)
    scr = _transpose_lut(lutT, tail)
    scr_rows = scr.reshape(VOCAB_SIZE, D)
    out4 = _make_gather(b_total, seq)(scr_rows, x)
    out5 = out4.reshape(seq, 8, b_total // 128, 8, 128)
    return jnp.transpose(out5, (2, 4, 0, 1, 3)).reshape(b_total, seq, D)


def kernel(x, lut):
    b0, b1 = x.shape
    assert b0 % NW == 0 and b1 % K2_RING == 0
    return _embed(x.astype(jnp.int32), lut)


# speedup vs baseline: 1.7899x; 1.7899x over previous
import functools
import jax
import jax.numpy as jnp
from jax import lax
from jax.experimental import pallas as pl
from jax.experimental.pallas import tpu as pltpu
from jax.experimental.pallas import tpu_sc as plsc

D = 64
SCALE = 8.0
NW = 32
VOCAB_SIZE = 1_000_000
BLK = 128
NBLK = VOCAB_SIZE // BLK
TAIL = VOCAB_SIZE - NBLK * BLK
K1_RING = 4
K2_RING = 4


def _iota16():
    return lax.iota(jnp.int32, 16)


@functools.partial(
    pl.kernel,
    out_type=jax.ShapeDtypeStruct((VOCAB_SIZE // 2, 2 * D), jnp.float32),
    mesh=plsc.VectorSubcoreMesh(core_axis_name="c", subcore_axis_name="s"),
    scratch_types=[
        [pltpu.VMEM((D, BLK), jnp.float32) for _ in range(K1_RING)],
        [pltpu.VMEM((BLK // 2, 2 * D), jnp.float32) for _ in range(2)],
        [pltpu.SemaphoreType.DMA for _ in range(K1_RING)],
        [pltpu.SemaphoreType.DMA for _ in range(2)],
    ],
    compiler_params=pltpu.CompilerParams(
        use_tc_tiling_on_sc=True, needs_layout_passes=False
    ),
)
def _transpose_lut(lutT_hbm, tail_hbm, scr_hbm, bufs, obufs, ssems, osems):
    wid = lax.axis_index("s") * 2 + lax.axis_index("c")
    per = NBLK // NW
    rem = NBLK - per * NW
    cnt = jnp.where(wid < rem, per + 1, per)
    lo = wid * per + jnp.minimum(wid, rem)
    riota = [_iota16() + 16 * m for m in range(4)]

    def fire_stage(u, b):
        pltpu.async_copy(
            lutT_hbm.at[:, pl.ds(u * BLK, BLK)], bufs[b], ssems[b]
        )

    def wait_stage(u, b):
        pltpu.make_async_copy(
            lutT_hbm.at[:, pl.ds(u * BLK, BLK)], bufs[b], ssems[b]
        ).wait()

    for p in range(K1_RING - 1):
        fire_stage(lo + p, p)

    n_slots = -(-(per + 1) // K1_RING) * K1_RING

    def body(i, carry):
        for b in range(K1_RING):
            u = i * K1_RING + b

            @pl.when(u < cnt)
            def _():
                blk = lo + u

                @pl.when(u + (K1_RING - 1) < cnt)
                def _():
                    fire_stage(blk + (K1_RING - 1), (b + K1_RING - 1) % K1_RING)

                wait_stage(blk, b)
                ob = b % 2

                @pl.when(u >= 2)
                def _():
                    pltpu.make_async_copy(
                        obufs[ob], scr_hbm.at[pl.ds(0, BLK // 2)], osems[ob]
                    ).wait()

                obuf = obufs[ob]
                buf = bufs[b]

                @plsc.parallel_loop(0, BLK // 2, unroll=4)
                def trans_body(q):
                    c0 = riota[0] * 0 + 2 * q
                    for k in range(8):
                        v = plsc.load_gather(buf, [riota[k % 4], c0 + (k // 4)])
                        obuf[q, pl.ds(16 * k, 16)] = v * SCALE
                pltpu.async_copy(
                    obuf, scr_hbm.at[pl.ds(blk * (BLK // 2), BLK // 2)], osems[ob]
                )
        return carry

    lax.fori_loop(0, n_slots // K1_RING, body, 0)
    for ob in range(2):
        pltpu.make_async_copy(
            obufs[ob], scr_hbm.at[pl.ds(0, BLK // 2)], osems[ob]
        ).wait()

    @pl.when(wid == NW - 1)
    def _():
        tb = obufs[0].at[pl.ds(0, TAIL // 2)]
        pltpu.sync_copy(tail_hbm, tb)

        def tail_body(q, c2):
            for k in range(8):
                sl = pl.ds(16 * k, 16)
                tb[q, sl] = tb[q, sl] * SCALE
            return c2

        lax.fori_loop(0, TAIL // 2, tail_body, 0)
        pltpu.sync_copy(tb, scr_hbm.at[pl.ds(NBLK * (BLK // 2), TAIL // 2)])


def _make_gather(b_total, seq):
    rows_per_worker = b_total // NW

    @functools.partial(
        pl.kernel,
        out_type=jax.ShapeDtypeStruct((seq, 8, b_total // 128, 8 * 128), jnp.float32),
        mesh=plsc.VectorSubcoreMesh(core_axis_name="c", subcore_axis_name="s"),
        scratch_types=[
            pltpu.VMEM((rows_per_worker, seq), jnp.int32),
            pltpu.VMEM((seq, rows_per_worker), jnp.int32),
            [pltpu.VMEM((rows_per_worker, D), jnp.float32) for _ in range(K2_RING)],
            [pltpu.VMEM((8, 8 * 128), jnp.float32) for _ in range(K2_RING)],
            [pltpu.SemaphoreType.DMA for _ in range(K2_RING)],
            [pltpu.SemaphoreType.DMA for _ in range(K2_RING)],
        ],
        compiler_params=pltpu.CompilerParams(
            use_tc_tiling_on_sc=False, needs_layout_passes=False
        ),
    )
    def k(scr_hbm, x_hbm, out_hbm, xbuf, idxT, gbufs, tbufs, gsems, osems):
        wid = lax.axis_index("s") * 2 + lax.axis_index("c")
        row0 = wid * rows_per_worker
        riota = [_iota16() + 16 * m for m in range(8)]

        pltpu.sync_copy(x_hbm.at[pl.ds(row0, rows_per_worker)], xbuf)

        @plsc.parallel_loop(0, seq, unroll=4)
        def xt_body(q):
            c0 = riota[0] * 0 + q
            for k in range(8):
                v = plsc.load_gather(xbuf, [riota[k], c0])
                idxT[q, pl.ds(16 * k, 16)] = v

        def fire_gather(c, b):
            pltpu.async_copy(scr_hbm.at[idxT.at[c]], gbufs[b], gsems[b])

        def drain_gather(c, b):
            pltpu.make_async_copy(
                scr_hbm.at[idxT.at[c]], gbufs[b], gsems[b]
            ).wait()

        PF = 2
        for b in range(PF):
            fire_gather(b, b)

        def body(i, carry):
            for b in range(K2_RING):
                c = i * K2_RING + b
                b_pre = (b + PF) % K2_RING

                @pl.when(c + PF <= seq - 1)
                def _():
                    fire_gather(c + PF, b_pre)

                drain_gather(c, b)
                gbuf = gbufs[b]
                tbuf = tbufs[b]

                @pl.when(c >= K2_RING)
                def _():
                    pltpu.make_async_copy(
                        tbufs[b], out_hbm.at[0, :, wid], osems[b]
                    ).wait()

                @plsc.parallel_loop(0, 8, unroll=2)
                def ext_body(dblk):
                    for d_in in range(8):
                        dcol = riota[0] * 0 + (dblk * 8 + d_in)
                        for g in range(8):
                            v = plsc.load_gather(gbuf, [riota[g], dcol])
                            tbuf[dblk, pl.ds(128 * d_in + 16 * g, 16)] = v
                pltpu.async_copy(tbuf, out_hbm.at[c, :, wid], osems[b])
            return carry

        lax.fori_loop(0, seq // K2_RING, body, 0)
        for b in range(K2_RING):
            pltpu.make_async_copy(
                tbufs[b], out_hbm.at[0, :, wid], osems[b]
            ).wait()

    return k


@jax.jit
def _embed(x, lut):
    b_total, seq = x.shape
    lutT = jnp.swapaxes(lut, 0, 1)
    tail = lut[NBLK * BLK :].reshape(TAIL // 2, 2 * D)
    scr = _transpose_lut(lutT, tail)
    scr_rows = scr.reshape(VOCAB_SIZE, D)
    out4 = _make_gather(b_total, seq)(scr_rows, x)
    out5 = out4.reshape(seq, 8, b_total // 128, 8, 128)
    return jnp.transpose(out5, (2, 4, 0, 1, 3)).reshape(b_total, seq, D)


def kernel(x, lut):
    b0, b1 = x.shape
    assert b0 % NW == 0 and b1 % K2_RING == 0
    return _embed(x.astype(jnp.int32), lut)
